# baseline (speedup 1.0000x reference)
import jax
import jax.numpy as jnp
from jax.experimental import pallas as pl
from jax.experimental.pallas import tpu as pltpu

_NUM_EXPERTS = 8
_TILE = 1024


def _moe_tile_kernel(x_ref, rw_ref, rb_ref, ew_hbm, eb_ref, out_ref,
                     wf_ref, wb_ref, sems):
    t = pl.program_id(0)

    @pl.when(t == 0)
    def _start_weight_copies():
        for ei in range(_NUM_EXPERTS):
            pltpu.make_async_copy(
                ew_hbm.at[ei], wf_ref.at[ei], sems.at[ei]
            ).start()

    x = x_ref[...]
    scores = (
        jnp.dot(x, rw_ref[...], preferred_element_type=jnp.float32)
        + rb_ref[...]
    )
    m = jnp.max(scores, axis=-1, keepdims=True)
    e = jnp.exp(scores - m)
    probs = e / jnp.sum(e, axis=-1, keepdims=True)

    i0 = jnp.argmax(probs, axis=-1).reshape(-1, 1)
    p0 = jnp.max(probs, axis=-1, keepdims=True)
    iota = jax.lax.broadcasted_iota(jnp.int32, probs.shape, 1)
    masked = jnp.where(iota == i0, probs - 2.0, probs)
    i1 = jnp.argmax(masked, axis=-1).reshape(-1, 1)
    p1 = jnp.max(masked, axis=-1, keepdims=True)

    denom = p0 + p1
    wmat = jnp.where(iota == i0, p0 / denom, 0.0) + jnp.where(
        iota == i1, p1 / denom, 0.0
    )

    xb = x.astype(jnp.bfloat16)
    acc = None
    for ei in range(_NUM_EXPERTS):
        @pl.when(t == 0)
        def _land_and_cast(ei=ei):
            pltpu.make_async_copy(
                ew_hbm.at[ei], wf_ref.at[ei], sems.at[ei]
            ).wait()
            wb_ref[ei] = wf_ref[ei].astype(jnp.bfloat16)

        w = wmat[:, ei].reshape(-1, 1)
        y = jnp.dot(xb, wb_ref[ei], preferred_element_type=jnp.float32)
        acc = w * y if acc is None else acc + w * y
    acc = acc + jnp.dot(wmat, eb_ref[...], preferred_element_type=jnp.float32)
    out_ref[...] = acc


def kernel(x, router_w, router_b, expert_w, expert_b):
    b, s, h = x.shape
    n_tok = b * s
    flat_x = x.reshape(n_tok, h)
    rb2 = router_b.reshape(1, -1)

    out = pl.pallas_call(
        _moe_tile_kernel,
        grid=(n_tok // _TILE,),
        in_specs=[
            pl.BlockSpec((_TILE, h), lambda t: (t, 0)),
            pl.BlockSpec((h, _NUM_EXPERTS), lambda t: (0, 0)),
            pl.BlockSpec((1, _NUM_EXPERTS), lambda t: (0, 0)),
            pl.BlockSpec(memory_space=pl.ANY),
            pl.BlockSpec((_NUM_EXPERTS, h), lambda t: (0, 0)),
        ],
        out_specs=pl.BlockSpec((_TILE, h), lambda t: (t, 0)),
        out_shape=jax.ShapeDtypeStruct((n_tok, h), jnp.float32),
        scratch_shapes=[
            pltpu.VMEM((_NUM_EXPERTS, h, h), jnp.float32),
            pltpu.VMEM((_NUM_EXPERTS, h, h), jnp.bfloat16),
            pltpu.SemaphoreType.DMA((_NUM_EXPERTS,)),
        ],
    )(flat_x, router_w, rb2, expert_w, expert_b)
    return out.reshape(b, s, h)


# speedup vs baseline: 1.2760x; 1.2760x over previous
import jax
import jax.numpy as jnp
from jax.experimental import pallas as pl
from jax.experimental.pallas import tpu as pltpu

_NUM_EXPERTS = 8
_TILE = 1024


def _moe_tile_kernel(x_ref, rw_ref, rb_ref, ew_ref, eb_ref, out_ref, wb_ref):
    @pl.when(pl.program_id(0) == 0)
    def _cast_weights():
        wb_ref[...] = ew_ref[...].astype(jnp.bfloat16)

    x = x_ref[...]
    scores = (
        jnp.dot(x, rw_ref[...], preferred_element_type=jnp.float32)
        + rb_ref[...]
    )
    m = jnp.max(scores, axis=-1, keepdims=True)
    e = jnp.exp(scores - m)
    probs = e / jnp.sum(e, axis=-1, keepdims=True)

    i0 = jnp.argmax(probs, axis=-1).reshape(-1, 1)
    p0 = jnp.max(probs, axis=-1, keepdims=True)
    iota = jax.lax.broadcasted_iota(jnp.int32, probs.shape, 1)
    masked = jnp.where(iota == i0, probs - 2.0, probs)
    i1 = jnp.argmax(masked, axis=-1).reshape(-1, 1)
    p1 = jnp.max(masked, axis=-1, keepdims=True)

    denom = p0 + p1
    wmat = jnp.where(iota == i0, p0 / denom, 0.0) + jnp.where(
        iota == i1, p1 / denom, 0.0
    )

    xb = x.astype(jnp.bfloat16)
    acc = None
    for ei in range(_NUM_EXPERTS):
        w = wmat[:, ei].reshape(-1, 1)
        y = jnp.dot(xb, wb_ref[ei], preferred_element_type=jnp.float32)
        acc = w * y if acc is None else acc + w * y
    acc = acc + jnp.dot(wmat, eb_ref[...], preferred_element_type=jnp.float32)
    out_ref[...] = acc


def kernel(x, router_w, router_b, expert_w, expert_b):
    b, s, h = x.shape
    n_tok = b * s
    flat_x = x.reshape(n_tok, h)
    rb2 = router_b.reshape(1, -1)

    out = pl.pallas_call(
        _moe_tile_kernel,
        grid=(n_tok // _TILE,),
        in_specs=[
            pl.BlockSpec((_TILE, h), lambda t: (t, 0)),
            pl.BlockSpec((h, _NUM_EXPERTS), lambda t: (0, 0)),
            pl.BlockSpec((1, _NUM_EXPERTS), lambda t: (0, 0)),
            pl.BlockSpec((_NUM_EXPERTS, h, h), lambda t: (0, 0, 0)),
            pl.BlockSpec((_NUM_EXPERTS, h), lambda t: (0, 0)),
        ],
        out_specs=pl.BlockSpec((_TILE, h), lambda t: (t, 0)),
        out_shape=jax.ShapeDtypeStruct((n_tok, h), jnp.float32),
        scratch_shapes=[
            pltpu.VMEM((_NUM_EXPERTS, h, h), jnp.bfloat16),
        ],
    )(flat_x, router_w, rb2, expert_w, expert_b)
    return out.reshape(b, s, h)
